# initial kernel scaffold (unmeasured)
import jax
import jax.numpy as jnp
from jax import lax
from jax.experimental import pallas as pl
from jax.experimental.pallas import tpu as pltpu

N_DEV = 4
EPS = 1e-5


def kernel(x, gamma):
    m, n_per = x.shape
    rows_div = 128
    m_outer = m // rows_div

    def body(x_ref, gamma_ref, out_ref, comm_ref, send_sems, recv_sems):
        my = lax.axis_index("i")

        xf = x_ref[...]
        x3 = xf.reshape(m_outer, rows_div, n_per)
        partial = jnp.sum(x3 * x3, axis=2)
        comm_ref[N_DEV - 1] = partial

        barrier_sem = pltpu.get_barrier_semaphore()
        for d in range(1, N_DEV):
            pl.semaphore_signal(
                barrier_sem,
                inc=1,
                device_id=((my + d) % N_DEV,),
                device_id_type=pl.DeviceIdType.MESH,
            )
        pl.semaphore_wait(barrier_sem, N_DEV - 1)

        rdmas = []
        for d in range(1, N_DEV):
            rdma = pltpu.make_async_remote_copy(
                src_ref=comm_ref.at[N_DEV - 1],
                dst_ref=comm_ref.at[N_DEV - 1 - d],
                send_sem=send_sems.at[d - 1],
                recv_sem=recv_sems.at[N_DEV - 1 - d],
                device_id=((my + d) % N_DEV,),
                device_id_type=pl.DeviceIdType.MESH,
            )
            rdma.start()
            rdmas.append(rdma)
        for rdma in rdmas:
            rdma.wait()

        total = comm_ref[0] + comm_ref[1] + comm_ref[2] + comm_ref[3]
        inv = lax.rsqrt(total * (1.0 / (N_DEV * n_per)) + EPS)

        g = gamma_ref[...]
        out3 = x3 * inv[:, :, None] * g[None, None, :]
        out_ref[...] = out3.reshape(m, n_per)

    return pl.pallas_call(
        body,
        out_shape=jax.ShapeDtypeStruct((m, n_per), x.dtype),
        in_specs=[
            pl.BlockSpec(memory_space=pltpu.VMEM),
            pl.BlockSpec(memory_space=pltpu.VMEM),
        ],
        out_specs=pl.BlockSpec(memory_space=pltpu.VMEM),
        scratch_shapes=[
            pltpu.VMEM((N_DEV, m_outer, rows_div), jnp.float32),
            pltpu.SemaphoreType.DMA((N_DEV - 1,)),
            pltpu.SemaphoreType.DMA((N_DEV - 1,)),
        ],
        compiler_params=pltpu.CompilerParams(collective_id=0),
    )(x, gamma)


# baseline (device time: 24547 ns/iter reference)
import jax
import jax.numpy as jnp
from jax import lax
from jax.experimental import pallas as pl
from jax.experimental.pallas import tpu as pltpu

N_DEV = 4
EPS = 1e-5


def kernel(x, gamma):
    m, n_per = x.shape
    rows_div = 128
    m_outer = m // rows_div

    def body(x_ref, gamma_ref, out_ref, comm_ref, send_sems, recv_sems):
        my = lax.axis_index("i")

        xf = x_ref[...]
        x3 = xf.reshape(m_outer, rows_div, n_per)
        partial = jnp.sum(x3 * x3, axis=2)
        comm_ref[N_DEV - 1] = partial

        barrier_sem = pltpu.get_barrier_semaphore()
        for d in range(1, N_DEV):
            pl.semaphore_signal(
                barrier_sem,
                inc=1,
                device_id=((my + d) % N_DEV,),
                device_id_type=pl.DeviceIdType.MESH,
            )
        pl.semaphore_wait(barrier_sem, N_DEV - 1)

        rdmas = []
        for d in range(1, N_DEV):
            rdma = pltpu.make_async_remote_copy(
                src_ref=comm_ref.at[N_DEV - 1],
                dst_ref=comm_ref.at[N_DEV - 1 - d],
                send_sem=send_sems.at[d - 1],
                recv_sem=recv_sems.at[N_DEV - 1 - d],
                device_id=((my + d) % N_DEV,),
                device_id_type=pl.DeviceIdType.MESH,
            )
            rdma.start()
            rdmas.append(rdma)
        for rdma in rdmas:
            rdma.wait()

        total = comm_ref[0] + comm_ref[1] + comm_ref[2] + comm_ref[3]
        inv = lax.rsqrt(total * (1.0 / (N_DEV * n_per)) + EPS)

        g = gamma_ref[...]
        out3 = x3 * inv[:, :, None] * g[None, None, :]
        out_ref[...] = out3.reshape(m, n_per).astype(jnp.bfloat16)

    return pl.pallas_call(
        body,
        out_shape=jax.ShapeDtypeStruct((m, n_per), jnp.bfloat16),
        in_specs=[
            pl.BlockSpec(memory_space=pltpu.VMEM),
            pl.BlockSpec(memory_space=pltpu.VMEM),
        ],
        out_specs=pl.BlockSpec(memory_space=pltpu.VMEM),
        scratch_shapes=[
            pltpu.VMEM((N_DEV, m_outer, rows_div), jnp.float32),
            pltpu.SemaphoreType.DMA((N_DEV - 1,)),
            pltpu.SemaphoreType.DMA((N_DEV - 1,)),
        ],
        compiler_params=pltpu.CompilerParams(
            collective_id=0, vmem_limit_bytes=64 * 1024 * 1024
        ),
    )(x, gamma)


# device time: 22324 ns/iter; 1.0996x vs baseline; 1.0996x over previous
import jax
import jax.numpy as jnp
from jax import lax
from jax.experimental import pallas as pl
from jax.experimental.pallas import tpu as pltpu

N_DEV = 4
EPS = 1e-5
N_CHUNK = 8


def kernel(x, gamma):
    m, n_per = x.shape
    rows_div = 128
    m_outer = m // rows_div
    bm = m // N_CHUNK
    bo = bm // rows_div

    def body(
        x_hbm,
        gamma_ref,
        out_hbm,
        xv,
        outv,
        comm_ref,
        load_sems,
        store_sems,
        send_sems,
        recv_sems,
    ):
        my = lax.axis_index("i")

        loads = []
        for c in range(N_CHUNK):
            cp = pltpu.make_async_copy(
                x_hbm.at[pl.ds(c * bm, bm), :], xv.at[c], load_sems.at[c]
            )
            cp.start()
            loads.append(cp)

        barrier_sem = pltpu.get_barrier_semaphore()
        for d in range(1, N_DEV):
            pl.semaphore_signal(
                barrier_sem,
                inc=1,
                device_id=((my + d) % N_DEV,),
                device_id_type=pl.DeviceIdType.MESH,
            )
        pl.semaphore_wait(barrier_sem, N_DEV - 1)

        for c in range(N_CHUNK):
            loads[c].wait()
            xc = xv[c].reshape(bo, rows_div, n_per)
            comm_ref[N_DEV - 1, pl.ds(c * bo, bo), :] = jnp.sum(xc * xc, axis=2)

        rdmas = []
        for d in range(1, N_DEV):
            rdma = pltpu.make_async_remote_copy(
                src_ref=comm_ref.at[N_DEV - 1],
                dst_ref=comm_ref.at[N_DEV - 1 - d],
                send_sem=send_sems.at[d - 1],
                recv_sem=recv_sems.at[N_DEV - 1 - d],
                device_id=((my + d) % N_DEV,),
                device_id_type=pl.DeviceIdType.MESH,
            )
            rdma.start()
            rdmas.append(rdma)
        for rdma in rdmas:
            rdma.wait()

        total = comm_ref[0] + comm_ref[1] + comm_ref[2] + comm_ref[3]
        inv = lax.rsqrt(total * (1.0 / (N_DEV * n_per)) + EPS)
        g = gamma_ref[...]

        stores = []
        for c in range(N_CHUNK):
            xc = xv[c].reshape(bo, rows_div, n_per)
            inv_c = inv[c * bo : (c + 1) * bo, :]
            oc = xc * inv_c[:, :, None] * g[None, None, :]
            outv[c] = oc.reshape(bm, n_per).astype(jnp.bfloat16)
            st = pltpu.make_async_copy(
                outv.at[c], out_hbm.at[pl.ds(c * bm, bm), :], store_sems.at[c]
            )
            st.start()
            stores.append(st)
        for st in stores:
            st.wait()

    return pl.pallas_call(
        body,
        out_shape=jax.ShapeDtypeStruct((m, n_per), jnp.bfloat16),
        in_specs=[
            pl.BlockSpec(memory_space=pl.ANY),
            pl.BlockSpec(memory_space=pltpu.VMEM),
        ],
        out_specs=pl.BlockSpec(memory_space=pl.ANY),
        scratch_shapes=[
            pltpu.VMEM((N_CHUNK, bm, n_per), jnp.float32),
            pltpu.VMEM((N_CHUNK, bm, n_per), jnp.bfloat16),
            pltpu.VMEM((N_DEV, m_outer, rows_div), jnp.float32),
            pltpu.SemaphoreType.DMA((N_CHUNK,)),
            pltpu.SemaphoreType.DMA((N_CHUNK,)),
            pltpu.SemaphoreType.DMA((N_DEV - 1,)),
            pltpu.SemaphoreType.DMA((N_DEV - 1,)),
        ],
        compiler_params=pltpu.CompilerParams(
            collective_id=0, vmem_limit_bytes=64 * 1024 * 1024
        ),
    )(x, gamma)


# device time: 20352 ns/iter; 1.2061x vs baseline; 1.0969x over previous
import jax
import jax.numpy as jnp
from jax import lax
from jax.experimental import pallas as pl
from jax.experimental.pallas import tpu as pltpu

N_DEV = 4
EPS = 1e-5
N_CHUNK = 8
N_HALF = 8
CHUNK_PER_HALF = N_CHUNK // N_HALF


def kernel(x, gamma):
    m, n_per = x.shape
    rows_div = 128
    m_outer = m // rows_div
    bm = m // N_CHUNK
    bo = bm // rows_div
    ho = m_outer // N_HALF

    def body(
        x_hbm,
        gamma_ref,
        out_hbm,
        xv,
        outv,
        comm_ref,
        load_sems,
        store_sems,
        send_sems,
        recv_sems,
    ):
        my = lax.axis_index("i")

        loads = []
        for c in range(N_CHUNK):
            cp = pltpu.make_async_copy(
                x_hbm.at[pl.ds(c * bm, bm), :], xv.at[c], load_sems.at[c]
            )
            cp.start()
            loads.append(cp)

        barrier_sem = pltpu.get_barrier_semaphore()
        for d in range(1, N_DEV):
            pl.semaphore_signal(
                barrier_sem,
                inc=1,
                device_id=((my + d) % N_DEV,),
                device_id_type=pl.DeviceIdType.MESH,
            )
        pl.semaphore_wait(barrier_sem, N_DEV - 1)

        rdmas = [[] for _ in range(N_HALF)]
        for h in range(N_HALF):
            for c in range(h * CHUNK_PER_HALF, (h + 1) * CHUNK_PER_HALF):
                loads[c].wait()
                xc = xv[c].reshape(bo, rows_div, n_per)
                comm_ref[N_DEV - 1, pl.ds(c * bo, bo), :] = jnp.sum(
                    xc * xc, axis=2
                )
            for d in range(1, N_DEV):
                rdma = pltpu.make_async_remote_copy(
                    src_ref=comm_ref.at[N_DEV - 1, pl.ds(h * ho, ho)],
                    dst_ref=comm_ref.at[N_DEV - 1 - d, pl.ds(h * ho, ho)],
                    send_sem=send_sems.at[d - 1, h],
                    recv_sem=recv_sems.at[N_DEV - 1 - d, h],
                    device_id=((my + d) % N_DEV,),
                    device_id_type=pl.DeviceIdType.MESH,
                )
                rdma.start()
                rdmas[h].append(rdma)

        g = gamma_ref[...]

        stores = []
        for h in range(N_HALF):
            for rdma in rdmas[h]:
                rdma.wait_recv()
            r0, r1 = h * ho, (h + 1) * ho
            total = (
                comm_ref[0, r0:r1]
                + comm_ref[1, r0:r1]
                + comm_ref[2, r0:r1]
                + comm_ref[3, r0:r1]
            )
            inv = lax.rsqrt(total * (1.0 / (N_DEV * n_per)) + EPS)
            for c in range(h * CHUNK_PER_HALF, (h + 1) * CHUNK_PER_HALF):
                xc = xv[c].reshape(bo, rows_div, n_per)
                k = c - h * CHUNK_PER_HALF
                inv_c = inv[k * bo : (k + 1) * bo, :]
                oc = xc * inv_c[:, :, None] * g[None, None, :]
                outv[c] = oc.reshape(bm, n_per).astype(jnp.bfloat16)
                st = pltpu.make_async_copy(
                    outv.at[c], out_hbm.at[pl.ds(c * bm, bm), :], store_sems.at[c]
                )
                st.start()
                stores.append(st)

        for h in range(N_HALF):
            for rdma in rdmas[h]:
                rdma.wait_send()
        for st in stores:
            st.wait()

    return pl.pallas_call(
        body,
        out_shape=jax.ShapeDtypeStruct((m, n_per), jnp.bfloat16),
        in_specs=[
            pl.BlockSpec(memory_space=pl.ANY),
            pl.BlockSpec(memory_space=pltpu.VMEM),
        ],
        out_specs=pl.BlockSpec(memory_space=pl.ANY),
        scratch_shapes=[
            pltpu.VMEM((N_CHUNK, bm, n_per), jnp.float32),
            pltpu.VMEM((N_CHUNK, bm, n_per), jnp.bfloat16),
            pltpu.VMEM((N_DEV, m_outer, rows_div), jnp.float32),
            pltpu.SemaphoreType.DMA((N_CHUNK,)),
            pltpu.SemaphoreType.DMA((N_CHUNK,)),
            pltpu.SemaphoreType.DMA((N_DEV - 1, N_HALF)),
            pltpu.SemaphoreType.DMA((N_DEV - 1, N_HALF)),
        ],
        compiler_params=pltpu.CompilerParams(
            collective_id=0, vmem_limit_bytes=64 * 1024 * 1024
        ),
    )(x, gamma)
